# baseline (speedup 1.0000x reference)
import functools

import jax
import jax.numpy as jnp
from jax import lax
from jax.experimental import pallas as pl
from jax.experimental.pallas import tpu as pltpu
from jax.experimental.pallas import tpu_sc as plsc

B = 16384
V = 1000001
G = 1001
D = 64
GL = 20

NC = 2
NS = 16
NW = NC * NS
BPW = B // NW
CHUNK = 128
NCHUNK = BPW // CHUNK
LANES = 16
NGRP = CHUNK // LANES


def _tree_sum(vals):
    vals = list(vals)
    while len(vals) > 1:
        nxt = [vals[i] + vals[i + 1] for i in range(0, len(vals) - 1, 2)]
        if len(vals) % 2:
            nxt.append(vals[-1])
        vals = nxt
    return vals[0]


def _sc_embed(item_id, genres_flat, item_table, gtab_flat):
    mesh = plsc.VectorSubcoreMesh(core_axis_name="c", subcore_axis_name="s")

    @functools.partial(
        pl.kernel,
        out_type=jax.ShapeDtypeStruct((2 * B, D), jnp.float32),
        mesh=mesh,
        compiler_params=pltpu.CompilerParams(
            needs_layout_passes=False, use_tc_tiling_on_sc=False),
        scratch_types=[
            pltpu.VMEM((G * D,), jnp.float32),
            pltpu.VMEM((NCHUNK, CHUNK), jnp.int32),
            pltpu.VMEM((CHUNK * GL,), jnp.int32),
            pltpu.VMEM((CHUNK, D), jnp.float32),
            pltpu.VMEM((CHUNK, D), jnp.float32),
            pltpu.SemaphoreType.DMA,
        ],
    )
    def k(item_id_hbm, genres_hbm, itab_hbm, gtab_hbm, emb_hbm,
          gt_v, ids_v, gids_v, irows_v, g_v, sem):
        wid = lax.axis_index("s") * NC + lax.axis_index("c")
        base = wid * BPW
        pltpu.sync_copy(gtab_hbm, gt_v)
        for c in range(NCHUNK):
            pltpu.sync_copy(item_id_hbm.at[pl.ds(base + c * CHUNK, CHUNK)],
                            ids_v.at[c])
        iota = lax.iota(jnp.int32, LANES)

        for c in range(NCHUNK):
            cbase = base + c * CHUNK
            item_gather = pltpu.async_copy(itab_hbm.at[ids_v.at[c]], irows_v, sem)
            pltpu.sync_copy(genres_hbm.at[pl.ds(cbase * GL, CHUNK * GL)], gids_v)

            def grp_body(g, _):
                row16 = g * LANES + iota
                rowg = row16 * GL
                ids = [plsc.load_gather(gids_v, [rowg + l]) for l in range(GL)]
                one = jnp.full((LANES,), 1.0, jnp.float32)
                zero = jnp.full((LANES,), 0.0, jnp.float32)
                cnts = [jnp.where(ids[l] > 0, one, zero) for l in range(GL)]
                cnt = _tree_sum(cnts)
                recip = 1.0 / (cnt + 1e-8)
                rowbase = [ids[l] * D for l in range(GL)]

                def d_body(dd, carry):
                    dvec = jnp.full((LANES,), dd, jnp.int32)
                    vals = [plsc.load_gather(gt_v, [rowbase[l] + dvec])
                            for l in range(GL)]
                    acc = _tree_sum(vals)
                    plsc.store_scatter(g_v, [row16, dvec], acc * recip)
                    return carry

                lax.fori_loop(0, D, d_body, 0)
                return _

            lax.fori_loop(0, NGRP, grp_body, 0)
            item_gather.wait()
            pltpu.sync_copy(irows_v, emb_hbm.at[pl.ds(cbase, CHUNK)])
            pltpu.sync_copy(g_v, emb_hbm.at[pl.ds(B + cbase, CHUNK)])

    return k(item_id, genres_flat, item_table, gtab_flat)


def _tc_mlp(i_emb, g_emb, cont8, wc8, bc, w1a, w1b, w1c, b1, w2, b2):
    BM = 1024
    grid = (B // BM,)

    def body(i_ref, g_ref, c_ref, wc_ref, bc_ref, w1a_ref, w1b_ref, w1c_ref,
             b1_ref, w2_ref, b2_ref, o_ref):
        ce = jnp.dot(c_ref[...], wc_ref[...], preferred_element_type=jnp.float32)
        ce = jnp.maximum(ce + bc_ref[...], 0.0)
        h = (jnp.dot(i_ref[...], w1a_ref[...], preferred_element_type=jnp.float32)
             + jnp.dot(g_ref[...], w1b_ref[...], preferred_element_type=jnp.float32)
             + jnp.dot(ce, w1c_ref[...], preferred_element_type=jnp.float32)
             + b1_ref[...])
        h = jnp.maximum(h, 0.0)
        out = jnp.dot(h, w2_ref[...], preferred_element_type=jnp.float32) + b2_ref[...]
        nrm = jnp.sqrt(jnp.sum(out * out, axis=1, keepdims=True))
        o_ref[...] = out / jnp.maximum(nrm, 1e-12)

    full = lambda shape: pl.BlockSpec(shape, lambda i: (0, 0))
    return pl.pallas_call(
        body,
        grid=grid,
        in_specs=[
            pl.BlockSpec((BM, D), lambda i: (i, 0)),
            pl.BlockSpec((BM, D), lambda i: (i, 0)),
            pl.BlockSpec((BM, 8), lambda i: (i, 0)),
            full((8, D)),
            full((1, D)),
            full((D, 128)),
            full((D, 128)),
            full((D, 128)),
            full((1, 128)),
            full((128, D)),
            full((1, D)),
        ],
        out_specs=pl.BlockSpec((BM, D), lambda i: (i, 0)),
        out_shape=jax.ShapeDtypeStruct((B, D), jnp.float32),
    )(i_emb, g_emb, cont8, wc8, bc, w1a, w1b, w1c, b1, w2, b2)


def kernel(item_id, tmdb_genres, release_year, avg_rating, revenue,
           item_table, genre_table, w_cont, b_cont, w1, b1, w2, b2):
    item_id = item_id.astype(jnp.int32)
    genres_flat = tmdb_genres.astype(jnp.int32).reshape(-1)
    gtab_flat = genre_table.reshape(-1)

    emb = _sc_embed(item_id, genres_flat, item_table, gtab_flat)
    i_emb, g_emb = emb[:B], emb[B:]

    cont = jnp.stack([release_year, avg_rating, revenue], axis=1)
    cont8 = jnp.pad(cont, ((0, 0), (0, 5)))
    wc8 = jnp.pad(w_cont, ((0, 5), (0, 0)))
    w1a, w1b, w1c = w1[:D], w1[D:2 * D], w1[2 * D:]

    return _tc_mlp(i_emb, g_emb, cont8, wc8, b_cont.reshape(1, D),
                   w1a, w1b, w1c, b1.reshape(1, 128), w2, b2.reshape(1, D))


# speedup vs baseline: 1.0431x; 1.0431x over previous
import functools

import jax
import jax.numpy as jnp
from jax import lax
from jax.experimental import pallas as pl
from jax.experimental.pallas import tpu as pltpu
from jax.experimental.pallas import tpu_sc as plsc

B = 16384
V = 1000001
G = 1001
D = 64
GL = 20

NC = 2
NS = 16
NW = NC * NS
BPW = B // NW
CHUNK = 128
NCHUNK = BPW // CHUNK
LANES = 16
NGRP = CHUNK // LANES


def _tree_sum(vals):
    vals = list(vals)
    while len(vals) > 1:
        nxt = [vals[i] + vals[i + 1] for i in range(0, len(vals) - 1, 2)]
        if len(vals) % 2:
            nxt.append(vals[-1])
        vals = nxt
    return vals[0]


def _sc_embed(item_id, genres_flat, item_table, gtab_flat):
    mesh = plsc.VectorSubcoreMesh(core_axis_name="c", subcore_axis_name="s")

    @functools.partial(
        pl.kernel,
        out_type=jax.ShapeDtypeStruct((2 * B, D), jnp.float32),
        mesh=mesh,
        compiler_params=pltpu.CompilerParams(
            needs_layout_passes=False, use_tc_tiling_on_sc=False),
        scratch_types=[
            pltpu.VMEM((G * D,), jnp.float32),
            pltpu.VMEM((NCHUNK, CHUNK), jnp.int32),
            pltpu.VMEM((CHUNK * GL,), jnp.int32),
            pltpu.VMEM((CHUNK, D), jnp.float32),
            pltpu.VMEM((CHUNK, D), jnp.float32),
            pltpu.SemaphoreType.DMA,
        ],
    )
    def k(item_id_hbm, genres_hbm, itab_hbm, gtab_hbm, emb_hbm,
          gt_v, ids_v, gids_v, irows_v, g_v, sem):
        wid = lax.axis_index("s") * NC + lax.axis_index("c")
        base = wid * BPW
        pltpu.sync_copy(gtab_hbm, gt_v)
        for c in range(NCHUNK):
            pltpu.sync_copy(item_id_hbm.at[pl.ds(base + c * CHUNK, CHUNK)],
                            ids_v.at[c])
        iota = lax.iota(jnp.int32, LANES)

        for c in range(NCHUNK):
            cbase = base + c * CHUNK
            item_gather = pltpu.async_copy(itab_hbm.at[ids_v.at[c]], irows_v, sem)
            pltpu.sync_copy(genres_hbm.at[pl.ds(cbase * GL, CHUNK * GL)], gids_v)

            def grp_body(g, _):
                row16 = g * LANES + iota
                rowg = row16 * GL
                ids = [plsc.load_gather(gids_v, [rowg + l]) for l in range(GL)]
                one = jnp.full((LANES,), 1.0, jnp.float32)
                zero = jnp.full((LANES,), 0.0, jnp.float32)
                cnts = [jnp.where(ids[l] > 0, one, zero) for l in range(GL)]
                cnt = _tree_sum(cnts)
                recip = 1.0 / (cnt + 1e-8)
                rowbase = [ids[l] * D for l in range(GL)]

                @plsc.parallel_loop(0, D, step=1, unroll=4)
                def d_body(dd):
                    dvec = jnp.full((LANES,), dd, jnp.int32)
                    vals = [plsc.load_gather(gt_v, [rowbase[l] + dvec])
                            for l in range(GL)]
                    acc = _tree_sum(vals)
                    plsc.store_scatter(g_v, [row16, dvec], acc * recip)

                return _

            lax.fori_loop(0, NGRP, grp_body, 0)
            item_gather.wait()
            pltpu.sync_copy(irows_v, emb_hbm.at[pl.ds(cbase, CHUNK)])
            pltpu.sync_copy(g_v, emb_hbm.at[pl.ds(B + cbase, CHUNK)])

    return k(item_id, genres_flat, item_table, gtab_flat)


def _tc_mlp(i_emb, g_emb, cont8, wc8, bc, w1a, w1b, w1c, b1, w2, b2):
    BM = 1024
    grid = (B // BM,)

    def body(i_ref, g_ref, c_ref, wc_ref, bc_ref, w1a_ref, w1b_ref, w1c_ref,
             b1_ref, w2_ref, b2_ref, o_ref):
        ce = jnp.dot(c_ref[...], wc_ref[...], preferred_element_type=jnp.float32)
        ce = jnp.maximum(ce + bc_ref[...], 0.0)
        h = (jnp.dot(i_ref[...], w1a_ref[...], preferred_element_type=jnp.float32)
             + jnp.dot(g_ref[...], w1b_ref[...], preferred_element_type=jnp.float32)
             + jnp.dot(ce, w1c_ref[...], preferred_element_type=jnp.float32)
             + b1_ref[...])
        h = jnp.maximum(h, 0.0)
        out = jnp.dot(h, w2_ref[...], preferred_element_type=jnp.float32) + b2_ref[...]
        nrm = jnp.sqrt(jnp.sum(out * out, axis=1, keepdims=True))
        o_ref[...] = out / jnp.maximum(nrm, 1e-12)

    full = lambda shape: pl.BlockSpec(shape, lambda i: (0, 0))
    return pl.pallas_call(
        body,
        grid=grid,
        in_specs=[
            pl.BlockSpec((BM, D), lambda i: (i, 0)),
            pl.BlockSpec((BM, D), lambda i: (i, 0)),
            pl.BlockSpec((BM, 8), lambda i: (i, 0)),
            full((8, D)),
            full((1, D)),
            full((D, 128)),
            full((D, 128)),
            full((D, 128)),
            full((1, 128)),
            full((128, D)),
            full((1, D)),
        ],
        out_specs=pl.BlockSpec((BM, D), lambda i: (i, 0)),
        out_shape=jax.ShapeDtypeStruct((B, D), jnp.float32),
    )(i_emb, g_emb, cont8, wc8, bc, w1a, w1b, w1c, b1, w2, b2)


def kernel(item_id, tmdb_genres, release_year, avg_rating, revenue,
           item_table, genre_table, w_cont, b_cont, w1, b1, w2, b2):
    item_id = item_id.astype(jnp.int32)
    genres_flat = tmdb_genres.astype(jnp.int32).reshape(-1)
    gtab_flat = genre_table.reshape(-1)

    emb = _sc_embed(item_id, genres_flat, item_table, gtab_flat)
    i_emb, g_emb = emb[:B], emb[B:]

    cont = jnp.stack([release_year, avg_rating, revenue], axis=1)
    cont8 = jnp.pad(cont, ((0, 0), (0, 5)))
    wc8 = jnp.pad(w_cont, ((0, 5), (0, 0)))
    w1a, w1b, w1c = w1[:D], w1[D:2 * D], w1[2 * D:]

    return _tc_mlp(i_emb, g_emb, cont8, wc8, b_cont.reshape(1, D),
                   w1a, w1b, w1c, b1.reshape(1, 128), w2, b2.reshape(1, D))


# speedup vs baseline: 1.3464x; 1.2908x over previous
import functools

import jax
import jax.numpy as jnp
from jax import lax
from jax.experimental import pallas as pl
from jax.experimental.pallas import tpu as pltpu
from jax.experimental.pallas import tpu_sc as plsc

B = 16384
V = 1000001
G = 1001
D = 64
GL = 20

NC = 2
NS = 16
NW = NC * NS
BPW = B // NW
CHUNK = 128
NCHUNK = BPW // CHUNK
LANES = 16
NGRP = CHUNK // LANES


def _tree_sum(vals):
    vals = list(vals)
    while len(vals) > 1:
        nxt = [vals[i] + vals[i + 1] for i in range(0, len(vals) - 1, 2)]
        if len(vals) % 2:
            nxt.append(vals[-1])
        vals = nxt
    return vals[0]


def _sc_embed(item_id, genres, item_table, genre_table):
    mesh = plsc.VectorSubcoreMesh(core_axis_name="c", subcore_axis_name="s")

    @functools.partial(
        pl.kernel,
        out_type=jax.ShapeDtypeStruct((2 * B, D), jnp.float32),
        mesh=mesh,
        compiler_params=pltpu.CompilerParams(
            needs_layout_passes=False, use_tc_tiling_on_sc=False),
        scratch_types=[
            pltpu.VMEM((G, D + 1), jnp.float32),
            pltpu.VMEM((NCHUNK, CHUNK), jnp.int32),
            pltpu.VMEM((CHUNK * GL,), jnp.int32),
            pltpu.VMEM((CHUNK, D), jnp.float32),
            pltpu.VMEM((CHUNK, D + 1), jnp.float32),
            pltpu.SemaphoreType.DMA,
        ],
    )
    def k(item_id_hbm, genres_hbm, itab_hbm, gtab_hbm, emb_hbm,
          gt_v, ids_v, gids_v, irows_v, g_v, sem):
        wid = lax.axis_index("s") * NC + lax.axis_index("c")
        base = wid * BPW
        pltpu.sync_copy(gtab_hbm, gt_v.at[:, pl.ds(0, D)])
        for c in range(NCHUNK):
            pltpu.sync_copy(item_id_hbm.at[pl.ds(base + c * CHUNK, CHUNK)],
                            ids_v.at[c])
        iota = lax.iota(jnp.int32, LANES)

        for c in range(NCHUNK):
            cbase = base + c * CHUNK
            item_gather = pltpu.async_copy(itab_hbm.at[ids_v.at[c]], irows_v, sem)
            pltpu.sync_copy(genres_hbm.at[pl.ds(cbase * GL, CHUNK * GL)], gids_v)

            def grp_body(g, _):
                row16 = g * LANES + iota
                rowg = row16 * GL
                ids = [plsc.load_gather(gids_v, [rowg + l]) for l in range(GL)]
                one = jnp.full((LANES,), 1.0, jnp.float32)
                zero = jnp.full((LANES,), 0.0, jnp.float32)
                cnts = [jnp.where(ids[l] > 0, one, zero) for l in range(GL)]
                cnt = _tree_sum(cnts)
                recip = 1.0 / (cnt + 1e-8)

                @plsc.parallel_loop(0, D, step=1, unroll=4)
                def d_body(dd):
                    dvec = jnp.full((LANES,), dd, jnp.int32)
                    vals = [plsc.load_gather(gt_v, [ids[l], dvec])
                            for l in range(GL)]
                    acc = _tree_sum(vals)
                    plsc.store_scatter(g_v, [row16, dvec], acc * recip)

                return _

            lax.fori_loop(0, NGRP, grp_body, 0)
            item_gather.wait()
            pltpu.sync_copy(irows_v, emb_hbm.at[pl.ds(cbase, CHUNK)])
            pltpu.sync_copy(g_v.at[:, pl.ds(0, D)], emb_hbm.at[pl.ds(B + cbase, CHUNK)])

    return k(item_id, genres, item_table, genre_table)


def _tc_mlp(i_emb, g_emb, cont8, wc8, bc, w1a, w1b, w1c, b1, w2, b2):
    BM = 1024
    grid = (B // BM,)

    def body(i_ref, g_ref, c_ref, wc_ref, bc_ref, w1a_ref, w1b_ref, w1c_ref,
             b1_ref, w2_ref, b2_ref, o_ref):
        ce = jnp.dot(c_ref[...], wc_ref[...], preferred_element_type=jnp.float32)
        ce = jnp.maximum(ce + bc_ref[...], 0.0)
        h = (jnp.dot(i_ref[...], w1a_ref[...], preferred_element_type=jnp.float32)
             + jnp.dot(g_ref[...], w1b_ref[...], preferred_element_type=jnp.float32)
             + jnp.dot(ce, w1c_ref[...], preferred_element_type=jnp.float32)
             + b1_ref[...])
        h = jnp.maximum(h, 0.0)
        out = jnp.dot(h, w2_ref[...], preferred_element_type=jnp.float32) + b2_ref[...]
        nrm = jnp.sqrt(jnp.sum(out * out, axis=1, keepdims=True))
        o_ref[...] = out / jnp.maximum(nrm, 1e-12)

    full = lambda shape: pl.BlockSpec(shape, lambda i: (0, 0))
    return pl.pallas_call(
        body,
        grid=grid,
        in_specs=[
            pl.BlockSpec((BM, D), lambda i: (i, 0)),
            pl.BlockSpec((BM, D), lambda i: (i, 0)),
            pl.BlockSpec((BM, 8), lambda i: (i, 0)),
            full((8, D)),
            full((1, D)),
            full((D, 128)),
            full((D, 128)),
            full((D, 128)),
            full((1, 128)),
            full((128, D)),
            full((1, D)),
        ],
        out_specs=pl.BlockSpec((BM, D), lambda i: (i, 0)),
        out_shape=jax.ShapeDtypeStruct((B, D), jnp.float32),
    )(i_emb, g_emb, cont8, wc8, bc, w1a, w1b, w1c, b1, w2, b2)


def kernel(item_id, tmdb_genres, release_year, avg_rating, revenue,
           item_table, genre_table, w_cont, b_cont, w1, b1, w2, b2):
    item_id = item_id.astype(jnp.int32)
    genres_flat = tmdb_genres.astype(jnp.int32).reshape(-1)

    emb = _sc_embed(item_id, genres_flat, item_table, genre_table)
    i_emb, g_emb = emb[:B], emb[B:]

    cont = jnp.stack([release_year, avg_rating, revenue], axis=1)
    cont8 = jnp.pad(cont, ((0, 0), (0, 5)))
    wc8 = jnp.pad(w_cont, ((0, 5), (0, 0)))
    w1a, w1b, w1c = w1[:D], w1[D:2 * D], w1[2 * D:]

    return _tc_mlp(i_emb, g_emb, cont8, wc8, b_cont.reshape(1, D),
                   w1a, w1b, w1c, b1.reshape(1, 128), w2, b2.reshape(1, D))
